# baseline (speedup 1.0000x reference)
import functools

import jax
import jax.numpy as jnp
from jax import lax
from jax.experimental import pallas as pl
from jax.experimental.pallas import tpu as pltpu


def _route_body(x_ref, w1_ref, b1_ref, w2_ref, b2_ref, wexp_ref, bexp_ref,
                wtot_ref, bcomb_ref, acc_ref, *, n_spatial, top_k):
    t = pl.program_id(0)

    @pl.when(t == 0)
    def _init():
        acc_ref[...] = jnp.zeros_like(acc_ref)

    acc_ref[...] += jnp.sum(x_ref[...], axis=2)

    @pl.when(t == pl.num_programs(0) - 1)
    def _epilogue():
        B = acc_ref.shape[0]
        E, CC = wexp_ref.shape
        C = bexp_ref.shape[1]
        pooled = acc_ref[...] * (1.0 / n_spatial)
        h = jax.nn.gelu(jnp.dot(pooled, w1_ref[...],
                                preferred_element_type=jnp.float32)
                        + b1_ref[...][None, :])
        logits = jnp.dot(h, w2_ref[...],
                         preferred_element_type=jnp.float32) + b2_ref[...][None, :]

        iota = lax.broadcasted_iota(jnp.int32, (B, E), 1)
        rem = logits
        vals = []
        hots = []
        for _ in range(top_k):
            m = jnp.max(rem, axis=1, keepdims=True)
            first = jnp.min(jnp.where(rem == m, iota, E), axis=1,
                            keepdims=True)
            sel = iota == first
            vals.append(m)
            hots.append(sel)
            rem = jnp.where(sel, -jnp.inf, rem)
        top_vals = jnp.concatenate(vals, axis=1)
        ex = jnp.exp(top_vals - top_vals[:, 0:1])
        w = ex / jnp.sum(ex, axis=1, keepdims=True)
        onehot_w = jnp.zeros((B, E), jnp.float32)
        for k in range(top_k):
            onehot_w += w[:, k:k + 1] * hots[k].astype(jnp.float32)

        wcomb = jnp.dot(onehot_w, wexp_ref[...],
                        preferred_element_type=jnp.float32)
        diag = (lax.broadcasted_iota(jnp.int32, (B, CC), 1) % (C + 1)) == 0
        wtot_ref[...] = wcomb + diag.astype(jnp.float32)
        bcomb_ref[...] = jnp.dot(onehot_w, bexp_ref[...],
                                 preferred_element_type=jnp.float32)


def _mix_body(x_ref, wtot_ref, bcomb_ref, out_ref):
    w = wtot_ref[0]
    out_ref[0] = (jnp.dot(w, x_ref[0], preferred_element_type=jnp.float32)
                  + bcomb_ref[0])


@jax.jit
def kernel(x, W1, b1, W2, b2, Wexp, bexp):
    B, C, H, Wd = x.shape
    E = Wexp.shape[0]
    top_k = 4
    HW = H * Wd
    xf = x.reshape(B, C, HW)

    S1 = 9216
    T1 = HW // S1
    wtot, bcomb = pl.pallas_call(
        functools.partial(_route_body, n_spatial=HW, top_k=top_k),
        grid=(T1,),
        in_specs=[
            pl.BlockSpec((B, C, S1), lambda t: (0, 0, t)),
            pl.BlockSpec((C, C // 4), lambda t: (0, 0)),
            pl.BlockSpec((C // 4,), lambda t: (0,)),
            pl.BlockSpec((C // 4, E), lambda t: (0, 0)),
            pl.BlockSpec((E,), lambda t: (0,)),
            pl.BlockSpec((E, C * C), lambda t: (0, 0)),
            pl.BlockSpec((E, C), lambda t: (0, 0)),
        ],
        out_specs=[
            pl.BlockSpec((B, C * C), lambda t: (0, 0)),
            pl.BlockSpec((B, C), lambda t: (0, 0)),
        ],
        out_shape=[
            jax.ShapeDtypeStruct((B, C * C), jnp.float32),
            jax.ShapeDtypeStruct((B, C), jnp.float32),
        ],
        scratch_shapes=[pltpu.VMEM((B, C), jnp.float32)],
    )(xf, W1, b1, W2, b2, Wexp.reshape(E, C * C), bexp)

    wtot = wtot.reshape(B, C, C)
    bcomb3 = bcomb.reshape(B, C, 1)

    S2 = 9216
    T2 = HW // S2
    out = pl.pallas_call(
        _mix_body,
        grid=(B, T2),
        in_specs=[
            pl.BlockSpec((1, C, S2), lambda b, t: (b, 0, t)),
            pl.BlockSpec((1, C, C), lambda b, t: (b, 0, 0)),
            pl.BlockSpec((1, C, 1), lambda b, t: (b, 0, 0)),
        ],
        out_specs=pl.BlockSpec((1, C, S2), lambda b, t: (b, 0, t)),
        out_shape=jax.ShapeDtypeStruct((B, C, HW), jnp.float32),
    )(xf, wtot, bcomb3)

    return out.reshape(B, C, H, Wd)


# speedup vs baseline: 2.5636x; 2.5636x over previous
import functools

import jax
import jax.numpy as jnp
from jax import lax
from jax.experimental import pallas as pl
from jax.experimental.pallas import tpu as pltpu

_HBLK_POOL = 48
_HBLK_MIX = 32


def _route_body(x_ref, w1_ref, b1_ref, w2_ref, b2_ref, wexp_ref, bexp_ref,
                wtot_ref, bcomb_ref, acc_ref, *, n_spatial, top_k):
    t = pl.program_id(0)

    @pl.when(t == 0)
    def _init():
        acc_ref[...] = jnp.zeros_like(acc_ref)

    acc_ref[...] += jnp.sum(x_ref[...], axis=(2, 3))

    @pl.when(t == pl.num_programs(0) - 1)
    def _epilogue():
        B = acc_ref.shape[0]
        E, CC = wexp_ref.shape
        C = bexp_ref.shape[1]
        pooled = acc_ref[...] * (1.0 / n_spatial)
        h = jax.nn.gelu(jnp.dot(pooled, w1_ref[...],
                                preferred_element_type=jnp.float32)
                        + b1_ref[...][None, :])
        logits = jnp.dot(h, w2_ref[...],
                         preferred_element_type=jnp.float32) + b2_ref[...][None, :]

        iota = lax.broadcasted_iota(jnp.int32, (B, E), 1)
        rem = logits
        vals = []
        hots = []
        for _ in range(top_k):
            m = jnp.max(rem, axis=1, keepdims=True)
            first = jnp.min(jnp.where(rem == m, iota, E), axis=1,
                            keepdims=True)
            sel = iota == first
            vals.append(m)
            hots.append(sel)
            rem = jnp.where(sel, -jnp.inf, rem)
        top_vals = jnp.concatenate(vals, axis=1)
        ex = jnp.exp(top_vals - top_vals[:, 0:1])
        w = ex / jnp.sum(ex, axis=1, keepdims=True)
        onehot_w = jnp.zeros((B, E), jnp.float32)
        for k in range(top_k):
            onehot_w += w[:, k:k + 1] * hots[k].astype(jnp.float32)

        wcomb = jnp.dot(onehot_w, wexp_ref[...],
                        preferred_element_type=jnp.float32)
        diag = (lax.broadcasted_iota(jnp.int32, (B, CC), 1) % (C + 1)) == 0
        wtot_ref[...] = wcomb + diag.astype(jnp.float32)
        bcomb_ref[...] = jnp.dot(onehot_w, bexp_ref[...],
                                 preferred_element_type=jnp.float32)


def _mix_body(x_ref, wtot_ref, bcomb_ref, out_ref, *, hblk):
    w = wtot_ref[0]
    b = bcomb_ref[0]
    for h in range(hblk):
        out_ref[0, :, h, :] = (
            jnp.dot(w, x_ref[0, :, h, :], preferred_element_type=jnp.float32)
            + b)


@jax.jit
def kernel(x, W1, b1, W2, b2, Wexp, bexp):
    B, C, H, Wd = x.shape
    E = Wexp.shape[0]
    top_k = 4
    HW = H * Wd

    T1 = H // _HBLK_POOL
    wtot, bcomb = pl.pallas_call(
        functools.partial(_route_body, n_spatial=HW, top_k=top_k),
        grid=(T1,),
        in_specs=[
            pl.BlockSpec((B, C, _HBLK_POOL, Wd), lambda t: (0, 0, t, 0)),
            pl.BlockSpec((C, C // 4), lambda t: (0, 0)),
            pl.BlockSpec((C // 4,), lambda t: (0,)),
            pl.BlockSpec((C // 4, E), lambda t: (0, 0)),
            pl.BlockSpec((E,), lambda t: (0,)),
            pl.BlockSpec((E, C * C), lambda t: (0, 0)),
            pl.BlockSpec((E, C), lambda t: (0, 0)),
        ],
        out_specs=[
            pl.BlockSpec((B, C * C), lambda t: (0, 0)),
            pl.BlockSpec((B, C), lambda t: (0, 0)),
        ],
        out_shape=[
            jax.ShapeDtypeStruct((B, C * C), jnp.float32),
            jax.ShapeDtypeStruct((B, C), jnp.float32),
        ],
        scratch_shapes=[pltpu.VMEM((B, C), jnp.float32)],
    )(x, W1, b1, W2, b2, Wexp.reshape(E, C * C), bexp)

    wtot = wtot.reshape(B, C, C)
    bcomb3 = bcomb.reshape(B, C, 1)

    T2 = H // _HBLK_MIX
    out = pl.pallas_call(
        functools.partial(_mix_body, hblk=_HBLK_MIX),
        grid=(B, T2),
        in_specs=[
            pl.BlockSpec((1, C, _HBLK_MIX, Wd), lambda b, t: (b, 0, t, 0)),
            pl.BlockSpec((1, C, C), lambda b, t: (b, 0, 0)),
            pl.BlockSpec((1, C, 1), lambda b, t: (b, 0, 0)),
        ],
        out_specs=pl.BlockSpec((1, C, _HBLK_MIX, Wd), lambda b, t: (b, 0, t, 0)),
        out_shape=jax.ShapeDtypeStruct((B, C, H, Wd), jnp.float32),
    )(x, wtot, bcomb3)

    return out


# speedup vs baseline: 2.6287x; 1.0254x over previous
import functools

import jax
import jax.numpy as jnp
from jax import lax
from jax.experimental import pallas as pl
from jax.experimental.pallas import tpu as pltpu

_HBLK_POOL = 64
_HBLK_MIX = 64


def _route_body(x_ref, w1_ref, b1_ref, w2_ref, b2_ref, wexp_ref, bexp_ref,
                wtot_ref, bcomb_ref, acc_ref, *, n_spatial, top_k):
    t = pl.program_id(0)

    @pl.when(t == 0)
    def _init():
        acc_ref[...] = jnp.zeros_like(acc_ref)

    acc_ref[...] += jnp.sum(x_ref[...], axis=(2, 3))

    @pl.when(t == pl.num_programs(0) - 1)
    def _epilogue():
        B = acc_ref.shape[0]
        E, CC = wexp_ref.shape
        C = bexp_ref.shape[1]
        pooled = acc_ref[...] * (1.0 / n_spatial)
        h = jax.nn.gelu(jnp.dot(pooled, w1_ref[...],
                                preferred_element_type=jnp.float32)
                        + b1_ref[...][None, :])
        logits = jnp.dot(h, w2_ref[...],
                         preferred_element_type=jnp.float32) + b2_ref[...][None, :]

        iota = lax.broadcasted_iota(jnp.int32, (B, E), 1)
        rem = logits
        vals = []
        hots = []
        for _ in range(top_k):
            m = jnp.max(rem, axis=1, keepdims=True)
            first = jnp.min(jnp.where(rem == m, iota, E), axis=1,
                            keepdims=True)
            sel = iota == first
            vals.append(m)
            hots.append(sel)
            rem = jnp.where(sel, -jnp.inf, rem)
        top_vals = jnp.concatenate(vals, axis=1)
        ex = jnp.exp(top_vals - top_vals[:, 0:1])
        w = ex / jnp.sum(ex, axis=1, keepdims=True)
        onehot_w = jnp.zeros((B, E), jnp.float32)
        for k in range(top_k):
            onehot_w += w[:, k:k + 1] * hots[k].astype(jnp.float32)

        wcomb = jnp.dot(onehot_w, wexp_ref[...],
                        preferred_element_type=jnp.float32)
        diag = (lax.broadcasted_iota(jnp.int32, (B, CC), 1) % (C + 1)) == 0
        wtot_ref[...] = wcomb + diag.astype(jnp.float32)
        bcomb_ref[...] = jnp.dot(onehot_w, bexp_ref[...],
                                 preferred_element_type=jnp.float32)


def _mix_body(x_ref, wtot_ref, bcomb_ref, out_ref, *, hblk):
    w = wtot_ref[0]
    b = bcomb_ref[0]
    for h in range(hblk):
        out_ref[0, :, h, :] = (
            jnp.dot(w, x_ref[0, :, h, :], preferred_element_type=jnp.float32)
            + b)


@jax.jit
def kernel(x, W1, b1, W2, b2, Wexp, bexp):
    B, C, H, Wd = x.shape
    E = Wexp.shape[0]
    top_k = 4
    HW = H * Wd

    T1 = H // _HBLK_POOL
    wtot, bcomb = pl.pallas_call(
        functools.partial(_route_body, n_spatial=HW, top_k=top_k),
        grid=(T1,),
        in_specs=[
            pl.BlockSpec((B, C, _HBLK_POOL, Wd), lambda t: (0, 0, t, 0)),
            pl.BlockSpec((C, C // 4), lambda t: (0, 0)),
            pl.BlockSpec((C // 4,), lambda t: (0,)),
            pl.BlockSpec((C // 4, E), lambda t: (0, 0)),
            pl.BlockSpec((E,), lambda t: (0,)),
            pl.BlockSpec((E, C * C), lambda t: (0, 0)),
            pl.BlockSpec((E, C), lambda t: (0, 0)),
        ],
        out_specs=[
            pl.BlockSpec((B, C * C), lambda t: (0, 0)),
            pl.BlockSpec((B, C), lambda t: (0, 0)),
        ],
        out_shape=[
            jax.ShapeDtypeStruct((B, C * C), jnp.float32),
            jax.ShapeDtypeStruct((B, C), jnp.float32),
        ],
        scratch_shapes=[pltpu.VMEM((B, C), jnp.float32)],
    )(x, W1, b1, W2, b2, Wexp.reshape(E, C * C), bexp)

    wtot = wtot.reshape(B, C, C)
    bcomb3 = bcomb.reshape(B, C, 1)

    T2 = H // _HBLK_MIX
    out = pl.pallas_call(
        functools.partial(_mix_body, hblk=_HBLK_MIX),
        grid=(B, T2),
        in_specs=[
            pl.BlockSpec((1, C, _HBLK_MIX, Wd), lambda b, t: (b, 0, t, 0)),
            pl.BlockSpec((1, C, C), lambda b, t: (b, 0, 0)),
            pl.BlockSpec((1, C, 1), lambda b, t: (b, 0, 0)),
        ],
        out_specs=pl.BlockSpec((1, C, _HBLK_MIX, Wd), lambda b, t: (b, 0, t, 0)),
        out_shape=jax.ShapeDtypeStruct((B, C, H, Wd), jnp.float32),
    )(x, wtot, bcomb3)

    return out


# speedup vs baseline: 2.7309x; 1.0389x over previous
import functools

import jax
import jax.numpy as jnp
from jax import lax
from jax.experimental import pallas as pl
from jax.experimental.pallas import tpu as pltpu

_HBLK = 48


def _fused_body(x_mean_ref, x_mix_ref, w1_ref, b1_ref, w2_ref, b2_ref,
                wexp_ref, bexpt_ref, out_ref, acc_ref, wtot_ref, bcomb_ref,
                *, n_spatial, top_k, n_batch, hblk):
    p = pl.program_id(0)
    t = pl.program_id(1)
    B = n_batch
    E, C, _ = wexp_ref.shape

    @pl.when((p == 0) & (t == 0))
    def _init():
        acc_ref[...] = jnp.zeros_like(acc_ref)

    sm = jnp.sum(x_mean_ref[...], axis=(2, 3))
    for bb in range(B):
        @pl.when(p == bb)
        def _acc(bb=bb):
            acc_ref[bb:bb + 1, :] += sm

    @pl.when((p >= 1) & (t == 0))
    def _route():
        pooled = jnp.zeros((1, C), jnp.float32)
        for bb in range(B):
            pooled = jnp.where(p - 1 == bb, acc_ref[bb:bb + 1, :], pooled)
        pooled = pooled * (1.0 / n_spatial)
        h = jax.nn.gelu(jnp.dot(pooled, w1_ref[...],
                                preferred_element_type=jnp.float32)
                        + b1_ref[...][None, :])
        logits = (jnp.dot(h, w2_ref[...], preferred_element_type=jnp.float32)
                  + b2_ref[...][None, :])

        iota = lax.broadcasted_iota(jnp.int32, (1, E), 1)
        rem = logits
        vals, firsts = [], []
        for _ in range(top_k):
            m = jnp.max(rem, axis=1, keepdims=True)
            first = jnp.min(jnp.where(rem == m, iota, E), axis=1,
                            keepdims=True)
            vals.append(m)
            firsts.append(first)
            rem = jnp.where(iota == first, -jnp.inf, rem)
        top_vals = jnp.concatenate(vals, axis=1)
        ex = jnp.exp(top_vals - top_vals[:, 0:1])
        w = ex / jnp.sum(ex, axis=1, keepdims=True)

        iota_e3 = lax.broadcasted_iota(jnp.int32, (E, 1, 1), 0)
        iota_e2 = lax.broadcasted_iota(jnp.int32, (E, 1), 0)
        oh3 = jnp.zeros((E, 1, 1), jnp.float32)
        oh2 = jnp.zeros((E, 1), jnp.float32)
        for k in range(top_k):
            wk = w[:, k:k + 1]
            oh3 += wk * (iota_e3 == firsts[k]).astype(jnp.float32)
            oh2 += wk * (iota_e2 == firsts[k]).astype(jnp.float32)

        wtot = jnp.sum(wexp_ref[...] * oh3, axis=0)
        eye = (lax.broadcasted_iota(jnp.int32, (C, C), 0)
               == lax.broadcasted_iota(jnp.int32, (C, C), 1))
        wtot_ref[...] = wtot + eye.astype(jnp.float32)
        bcomb_ref[...] = jnp.dot(bexpt_ref[...], oh2,
                                 preferred_element_type=jnp.float32)

    @pl.when(p >= 1)
    def _mix():
        wm = wtot_ref[...]
        bc = bcomb_ref[...]
        for h in range(hblk):
            out_ref[0, :, h, :] = (
                jnp.dot(wm, x_mix_ref[0, :, h, :],
                        preferred_element_type=jnp.float32) + bc)


@jax.jit
def kernel(x, W1, b1, W2, b2, Wexp, bexp):
    B, C, H, Wd = x.shape
    E = Wexp.shape[0]
    top_k = 4
    HW = H * Wd
    T = H // _HBLK

    def mean_idx(p, t):
        return (jnp.minimum(p, B - 1), 0,
                jnp.where(p < B, t, T - 1), 0)

    def mix_idx(p, t):
        return (jnp.maximum(p - 1, 0), 0,
                jnp.where(p >= 1, t, 0), 0)

    out = pl.pallas_call(
        functools.partial(_fused_body, n_spatial=HW, top_k=top_k,
                          n_batch=B, hblk=_HBLK),
        grid=(B + 1, T),
        in_specs=[
            pl.BlockSpec((1, C, _HBLK, Wd), mean_idx),
            pl.BlockSpec((1, C, _HBLK, Wd), mix_idx),
            pl.BlockSpec((C, C // 4), lambda p, t: (0, 0)),
            pl.BlockSpec((C // 4,), lambda p, t: (0,)),
            pl.BlockSpec((C // 4, E), lambda p, t: (0, 0)),
            pl.BlockSpec((E,), lambda p, t: (0,)),
            pl.BlockSpec((E, C, C), lambda p, t: (0, 0, 0)),
            pl.BlockSpec((C, E), lambda p, t: (0, 0)),
        ],
        out_specs=pl.BlockSpec((1, C, _HBLK, Wd), mix_idx),
        out_shape=jax.ShapeDtypeStruct((B, C, H, Wd), jnp.float32),
        scratch_shapes=[
            pltpu.VMEM((B, C), jnp.float32),
            pltpu.VMEM((C, C), jnp.float32),
            pltpu.VMEM((C, 1), jnp.float32),
        ],
    )(x, x, W1, b1, W2, b2, Wexp, bexp.T)

    return out
